# Initial kernel scaffold; baseline (speedup 1.0000x reference)
import functools

import jax
import jax.numpy as jnp
from jax import lax
from jax.experimental import pallas as pl
from jax.experimental.pallas import tpu as pltpu
from jax.experimental.pallas import tpu_sc as plsc

B = 16384
C = 5
E = 64
NC = 2
NS = 16
NW = NC * NS
BPW = B // NW
CHUNK = 128
NCHUNK = BPW // CHUNK
LANES = 16


def _sc_note2vec(tgt_idx, ctx_idx, target_table, context_table):
  mesh = plsc.VectorSubcoreMesh(core_axis_name="c", subcore_axis_name="s")

  @functools.partial(
      pl.kernel,
      out_type=jax.ShapeDtypeStruct((B, C), jnp.float32),
      mesh=mesh,
      scratch_types=[
          pltpu.VMEM((NCHUNK, CHUNK), jnp.int32),
          pltpu.VMEM((NCHUNK * C, CHUNK), jnp.int32),
          pltpu.VMEM((CHUNK, E), jnp.float32),
          pltpu.VMEM((CHUNK * C, E), jnp.float32),
          pltpu.VMEM((BPW, C), jnp.float32),
          pltpu.SemaphoreType.DMA,
      ],
  )
  def k(tgt_hbm, ctx_hbm, ttab_hbm, ctab_hbm, out_hbm,
        tidx_v, cidx_v, trows_v, crows_v, out_v, sem):
    wid = lax.axis_index("s") * NC + lax.axis_index("c")
    pltpu.sync_copy(tgt_hbm.at[pl.ds(wid * NCHUNK, NCHUNK)], tidx_v)
    pltpu.sync_copy(ctx_hbm.at[pl.ds(wid * NCHUNK * C, NCHUNK * C)], cidx_v)

    for j in range(NCHUNK):
      cps = [pltpu.async_copy(ttab_hbm.at[tidx_v.at[j]], trows_v, sem)]
      for i in range(C):
        cps.append(
            pltpu.async_copy(ctab_hbm.at[cidx_v.at[C * j + i]],
                             crows_v.at[pl.ds(i * CHUNK, CHUNK)], sem))
      for cp in cps:
        cp.wait()

      @pl.loop(0, CHUNK)
      def _(b):
        w = [trows_v[b, pl.ds(LANES * k, LANES)] for k in range(E // LANES)]
        for c in range(C):
          r = b * C + c
          acc = w[0] * crows_v[r, pl.ds(0, LANES)]
          for k in range(1, E // LANES):
            acc = acc + w[k] * crows_v[r, pl.ds(LANES * k, LANES)]
          out_v[j * CHUNK + b, c] = jnp.sum(acc)

    pltpu.sync_copy(out_v, out_hbm.at[pl.ds(wid * BPW, BPW)])

  return k(tgt_idx, ctx_idx, target_table, context_table)


def kernel(target, context, target_table, context_table):
  tgt_idx = target.reshape(-1).astype(jnp.int32).reshape(NW * NCHUNK, CHUNK)
  ctx_idx = context.reshape(-1).astype(jnp.int32).reshape(NW * NCHUNK * C, CHUNK)
  return _sc_note2vec(tgt_idx, ctx_idx, target_table, context_table)


# speedup vs baseline: 2.7031x; 2.7031x over previous
import dataclasses
import functools

import jax
import jax.numpy as jnp
from jax import lax
from jax.experimental import pallas as pl
from jax.experimental.pallas import tpu as pltpu
from jax.experimental.pallas import tpu_sc as plsc

B = 16384
C = 5
E = 64
NC = 2
NS = 16
NW = NC * NS
BPW = B // NW
CHUNK = 128
NCHUNK = BPW // CHUNK
LANES = 16


def _sc_note2vec(tgt_idx, ctx_idx, target_table, context_table):
  mesh = plsc.VectorSubcoreMesh(core_axis_name="c", subcore_axis_name="s")

  cp = pltpu.CompilerParams()
  if "needs_layout_passes" in pltpu.CompilerParams.__dataclass_fields__:
    cp = dataclasses.replace(cp, needs_layout_passes=False)
  if "use_tc_tiling_on_sc" in pltpu.CompilerParams.__dataclass_fields__:
    cp = dataclasses.replace(cp, use_tc_tiling_on_sc=False)

  @functools.partial(
      pl.kernel,
      compiler_params=cp,
      out_type=jax.ShapeDtypeStruct((B * C,), jnp.float32),
      mesh=mesh,
      scratch_types=[
          pltpu.VMEM((NCHUNK, CHUNK), jnp.int32),
          pltpu.VMEM((NCHUNK * C, CHUNK), jnp.int32),
          pltpu.VMEM((CHUNK, E), jnp.float32),
          pltpu.VMEM((CHUNK * C, E), jnp.float32),
          pltpu.VMEM((BPW * C,), jnp.float32),
          pltpu.SemaphoreType.DMA,
      ],
  )
  def k(tgt_hbm, ctx_hbm, ttab_hbm, ctab_hbm, out_hbm,
        tidx_v, cidx_v, trows_v, crows_v, out_v, sem):
    wid = lax.axis_index("s") * NC + lax.axis_index("c")
    pltpu.sync_copy(tgt_hbm.at[wid], tidx_v)
    pltpu.sync_copy(ctx_hbm.at[wid], cidx_v)

    lane = lax.iota(jnp.int32, LANES)
    cmask = [lane == c for c in range(C)]
    store_mask = lane < C

    for j in range(NCHUNK):
      cps = [pltpu.async_copy(ttab_hbm.at[tidx_v.at[j]], trows_v, sem)]
      for i in range(C):
        cps.append(
            pltpu.async_copy(ctab_hbm.at[cidx_v.at[C * j + i]],
                             crows_v.at[pl.ds(i * CHUNK, CHUNK)], sem))
      for cp in cps:
        cp.wait()

      @pl.loop(0, CHUNK)
      def _(b):
        w = [trows_v[b, pl.ds(LANES * k, LANES)] for k in range(E // LANES)]
        dots = jnp.zeros((LANES,), jnp.float32)
        for c in range(C):
          r = b * C + c
          acc = w[0] * crows_v[r, pl.ds(0, LANES)]
          for k in range(1, E // LANES):
            acc = acc + w[k] * crows_v[r, pl.ds(LANES * k, LANES)]
          dots = jnp.where(cmask[c], jnp.sum(acc), dots)
        plsc.store_scatter(out_v, [(j * CHUNK + b) * C + lane], dots,
                           mask=store_mask)

    pltpu.sync_copy(out_v, out_hbm.at[pl.ds(wid * BPW * C, BPW * C)])

  return k(tgt_idx, ctx_idx, target_table, context_table)


def kernel(target, context, target_table, context_table):
  tgt_idx = target.reshape(-1).astype(jnp.int32).reshape(NW, NCHUNK, CHUNK)
  ctx_idx = context.reshape(-1).astype(jnp.int32).reshape(NW, NCHUNK * C, CHUNK)
  out = _sc_note2vec(tgt_idx, ctx_idx, target_table, context_table)
  return out.reshape(B, C)


# speedup vs baseline: 2.8831x; 1.0666x over previous
import dataclasses
import functools

import jax
import jax.numpy as jnp
from jax import lax
from jax.experimental import pallas as pl
from jax.experimental.pallas import tpu as pltpu
from jax.experimental.pallas import tpu_sc as plsc

B = 16384
C = 5
E = 64
NC = 2
NS = 16
NW = NC * NS
BPW = B // NW
CHUNK = 128
NCHUNK = BPW // CHUNK
LANES = 16


def _sc_note2vec(cat_table, tgt_idx, ctx_idx):
  mesh = plsc.VectorSubcoreMesh(core_axis_name="c", subcore_axis_name="s")

  cp = pltpu.CompilerParams()
  if "needs_layout_passes" in pltpu.CompilerParams.__dataclass_fields__:
    cp = dataclasses.replace(cp, needs_layout_passes=False)

  @functools.partial(
      pl.kernel,
      out_type=jax.ShapeDtypeStruct((B * C,), jnp.float32),
      mesh=mesh,
      compiler_params=cp,
      scratch_types=[
          pltpu.VMEM((BPW,), jnp.int32),
          pltpu.VMEM((BPW * C,), jnp.int32),
          pltpu.VMEM((CHUNK, 2 * E), jnp.float32),
          pltpu.VMEM((CHUNK * C, 2 * E), jnp.float32),
          pltpu.VMEM((BPW * C,), jnp.float32),
          pltpu.SemaphoreType.DMA,
      ],
  )
  def k(tab_hbm, tgt_hbm, ctx_hbm, out_hbm,
        tidx_v, cidx_v, trows_v, crows_v, out_v, sem):
    wid = lax.axis_index("s") * NC + lax.axis_index("c")
    pltpu.sync_copy(tgt_hbm.at[pl.ds(wid * BPW, BPW)], tidx_v)
    pltpu.sync_copy(ctx_hbm.at[pl.ds(wid * BPW * C, BPW * C)], cidx_v)

    lane = lax.iota(jnp.int32, LANES)
    cmask = [lane == c for c in range(C)]
    store_mask = lane < C

    for j in range(NCHUNK):
      cps = [pltpu.async_copy(
          tab_hbm.at[tidx_v.at[pl.ds(j * CHUNK, CHUNK)]], trows_v, sem)]
      for i in range(C):
        cps.append(
            pltpu.async_copy(
                tab_hbm.at[cidx_v.at[pl.ds((j * C + i) * CHUNK, CHUNK)]],
                crows_v.at[pl.ds(i * CHUNK, CHUNK)], sem))
      for cp_ in cps:
        cp_.wait()

      @pl.loop(0, CHUNK)
      def _(b):
        w = [trows_v[b, pl.ds(LANES * k, LANES)] for k in range(E // LANES)]
        dots = jnp.zeros((LANES,), jnp.float32)
        for c in range(C):
          r = b * C + c
          acc = w[0] * crows_v[r, pl.ds(E, LANES)]
          for k in range(1, E // LANES):
            acc = acc + w[k] * crows_v[r, pl.ds(E + LANES * k, LANES)]
          dots = jnp.where(cmask[c], jnp.sum(acc), dots)
        plsc.store_scatter(out_v, [(j * CHUNK + b) * C + lane], dots,
                           mask=store_mask)

    pltpu.sync_copy(out_v, out_hbm.at[pl.ds(wid * BPW * C, BPW * C)])

  return k(cat_table, tgt_idx, ctx_idx)


def kernel(target, context, target_table, context_table):
  cat_table = jnp.concatenate([target_table, context_table], axis=1)
  tgt_idx = target.reshape(-1).astype(jnp.int32)
  ctx_idx = context.reshape(-1).astype(jnp.int32)
  out = _sc_note2vec(cat_table, tgt_idx, ctx_idx)
  return out.reshape(B, C)


# speedup vs baseline: 3.6761x; 1.2751x over previous
import dataclasses
import functools

import jax
import jax.numpy as jnp
from jax import lax
from jax.experimental import pallas as pl
from jax.experimental.pallas import tpu as pltpu
from jax.experimental.pallas import tpu_sc as plsc

B = 16384
C = 5
E = 64
NC = 2
NS = 16
NW = NC * NS
BPW = B // NW
CHUNK = 128
NCHUNK = BPW // CHUNK
LANES = 16


def _sc_note2vec(t_table, c_table, tgt_idx, ctx_idx):
  mesh = plsc.VectorSubcoreMesh(core_axis_name="c", subcore_axis_name="s")

  cp = pltpu.CompilerParams()
  if "needs_layout_passes" in pltpu.CompilerParams.__dataclass_fields__:
    cp = dataclasses.replace(cp, needs_layout_passes=False)

  @functools.partial(
      pl.kernel,
      out_type=jax.ShapeDtypeStruct((B * C,), jnp.float32),
      mesh=mesh,
      compiler_params=cp,
      scratch_types=[
          pltpu.VMEM((BPW,), jnp.int32),
          pltpu.VMEM((BPW * C,), jnp.int32),
          pltpu.VMEM((CHUNK, 2 * E), jnp.float32),
          pltpu.VMEM((CHUNK * C, 2 * E), jnp.float32),
          pltpu.VMEM((BPW * C,), jnp.float32),
          pltpu.SemaphoreType.DMA,
      ],
  )
  def k(ttab_hbm, ctab_hbm, tgt_hbm, ctx_hbm, out_hbm,
        tidx_v, cidx_v, trows_v, crows_v, out_v, sem):
    wid = lax.axis_index("s") * NC + lax.axis_index("c")
    pltpu.sync_copy(tgt_hbm.at[pl.ds(wid * BPW, BPW)], tidx_v)
    pltpu.sync_copy(ctx_hbm.at[pl.ds(wid * BPW * C, BPW * C)], cidx_v)

    lane = lax.iota(jnp.int32, LANES)
    cmask = [lane == c for c in range(C)]
    store_mask = lane < C

    for j in range(NCHUNK):
      cps = [pltpu.async_copy(
          ttab_hbm.at[tidx_v.at[pl.ds(j * CHUNK, CHUNK)]], trows_v, sem)]
      for i in range(C):
        cps.append(
            pltpu.async_copy(
                ctab_hbm.at[cidx_v.at[pl.ds((j * C + i) * CHUNK, CHUNK)]],
                crows_v.at[pl.ds(i * CHUNK, CHUNK)], sem))
      for cp_ in cps:
        cp_.wait()

      @pl.loop(0, CHUNK)
      def _(b):
        w = [trows_v[b, pl.ds(LANES * k, LANES)] for k in range(E // LANES)]
        dots = jnp.zeros((LANES,), jnp.float32)
        for c in range(C):
          r = b * C + c
          acc = w[0] * crows_v[r, pl.ds(0, LANES)]
          for k in range(1, E // LANES):
            acc = acc + w[k] * crows_v[r, pl.ds(LANES * k, LANES)]
          dots = jnp.where(cmask[c], jnp.sum(acc), dots)
        plsc.store_scatter(out_v, [(j * CHUNK + b) * C + lane], dots,
                           mask=store_mask)

    pltpu.sync_copy(out_v, out_hbm.at[pl.ds(wid * BPW * C, BPW * C)])

  return k(t_table, c_table, tgt_idx, ctx_idx)


def kernel(target, context, target_table, context_table):
  eye = jnp.eye(E, 2 * E, dtype=jnp.float32)
  t_table = jnp.einsum("ve,ef->vf", target_table, eye,
                       preferred_element_type=jnp.float32)
  c_table = jnp.einsum("ve,ef->vf", context_table, eye,
                       preferred_element_type=jnp.float32)
  tgt_idx = target.reshape(-1).astype(jnp.int32)
  ctx_idx = context.reshape(-1).astype(jnp.int32)
  out = _sc_note2vec(t_table, c_table, tgt_idx, ctx_idx)
  return out.reshape(B, C)


# speedup vs baseline: 4.4865x; 1.2204x over previous
import dataclasses
import functools

import jax
import jax.numpy as jnp
from jax import lax
from jax.experimental import pallas as pl
from jax.experimental.pallas import tpu as pltpu
from jax.experimental.pallas import tpu_sc as plsc

B = 16384
C = 5
E = 64
NC = 2
NS = 16
NW = NC * NS
BPW = B // NW
CHUNK = 128
NCHUNK = BPW // CHUNK
LANES = 16


def _sc_note2vec(t_table, c_table, tgt_idx, ctx_idx):
  mesh = plsc.VectorSubcoreMesh(core_axis_name="c", subcore_axis_name="s")

  cp = pltpu.CompilerParams()
  if "needs_layout_passes" in pltpu.CompilerParams.__dataclass_fields__:
    cp = dataclasses.replace(cp, needs_layout_passes=False)

  @functools.partial(
      pl.kernel,
      out_type=jax.ShapeDtypeStruct((C * B,), jnp.float32),
      mesh=mesh,
      compiler_params=cp,
      scratch_types=[
          pltpu.VMEM((BPW,), jnp.int32),
          pltpu.VMEM((C * BPW,), jnp.int32),
          pltpu.VMEM((CHUNK, 2 * E), jnp.float32),
          pltpu.VMEM((C * CHUNK, 2 * E), jnp.float32),
          pltpu.VMEM((C * BPW,), jnp.float32),
          pltpu.SemaphoreType.DMA,
      ],
  )
  def k(ttab_hbm, ctab_hbm, tgt_hbm, ctx_hbm, out_hbm,
        tidx_v, cidx_v, trows_v, crows_v, out_v, sem):
    wid = lax.axis_index("s") * NC + lax.axis_index("c")
    pltpu.sync_copy(tgt_hbm.at[pl.ds(wid * BPW, BPW)], tidx_v)
    for c in range(C):
      pltpu.sync_copy(ctx_hbm.at[pl.ds(c * B + wid * BPW, BPW)],
                      cidx_v.at[pl.ds(c * BPW, BPW)])

    lane = lax.iota(jnp.int32, LANES)
    cmask = [lane == c for c in range(C)]
    store_mask = lane < C

    for j in range(NCHUNK):
      cps = [pltpu.async_copy(
          ttab_hbm.at[tidx_v.at[pl.ds(j * CHUNK, CHUNK)]], trows_v, sem)]
      for i in range(C):
        cps.append(
            pltpu.async_copy(
                ctab_hbm.at[cidx_v.at[pl.ds(i * BPW + j * CHUNK, CHUNK)]],
                crows_v.at[pl.ds(i * CHUNK, CHUNK)], sem))
      for cp_ in cps:
        cp_.wait()

      @pl.loop(0, CHUNK)
      def _(b):
        w = [trows_v[b, pl.ds(LANES * k, LANES)] for k in range(E // LANES)]
        dots = jnp.zeros((LANES,), jnp.float32)
        for c in range(C):
          r = c * CHUNK + b
          acc = w[0] * crows_v[r, pl.ds(0, LANES)]
          for k in range(1, E // LANES):
            acc = acc + w[k] * crows_v[r, pl.ds(LANES * k, LANES)]
          dots = jnp.where(cmask[c], jnp.sum(acc), dots)
        plsc.store_scatter(out_v, [lane * BPW + (j * CHUNK + b)], dots,
                           mask=store_mask)

    for c in range(C):
      pltpu.sync_copy(out_v.at[pl.ds(c * BPW, BPW)],
                      out_hbm.at[pl.ds(c * B + wid * BPW, BPW)])

  return k(t_table, c_table, tgt_idx, ctx_idx)


def kernel(target, context, target_table, context_table):
  eye = jnp.eye(E, 2 * E, dtype=jnp.float32)
  t_table = jnp.einsum("ve,ef->vf", target_table, eye,
                       preferred_element_type=jnp.float32)
  c_table = jnp.einsum("ve,ef->vf", context_table, eye,
                       preferred_element_type=jnp.float32)
  tgt_idx = target.reshape(-1).astype(jnp.int32)
  ctx_idx = context.T.reshape(-1).astype(jnp.int32)
  out = _sc_note2vec(t_table, c_table, tgt_idx, ctx_idx)
  return out.reshape(C, B).T


# speedup vs baseline: 4.9610x; 1.1057x over previous
import dataclasses
import functools

import jax
import jax.numpy as jnp
from jax import lax
from jax.experimental import pallas as pl
from jax.experimental.pallas import tpu as pltpu
from jax.experimental.pallas import tpu_sc as plsc

B = 16384
C = 5
E = 64
NC = 2
NS = 16
NW = NC * NS
BPW = B // NW
CHUNK = 64
NCHUNK = BPW // CHUNK
NBUF = 2
LANES = 16


def _sc_note2vec(t_table, c_table, tgt_idx, ctx_idx):
  mesh = plsc.VectorSubcoreMesh(core_axis_name="c", subcore_axis_name="s")

  cp = pltpu.CompilerParams()
  if "needs_layout_passes" in pltpu.CompilerParams.__dataclass_fields__:
    cp = dataclasses.replace(cp, needs_layout_passes=False)

  @functools.partial(
      pl.kernel,
      out_type=jax.ShapeDtypeStruct((C * B,), jnp.float32),
      mesh=mesh,
      compiler_params=cp,
      scratch_types=[
          pltpu.VMEM((BPW,), jnp.int32),
          pltpu.VMEM((C * BPW,), jnp.int32),
          pltpu.VMEM((NBUF, CHUNK, 2 * E), jnp.float32),
          pltpu.VMEM((NBUF, C * CHUNK, 2 * E), jnp.float32),
          pltpu.VMEM((C * BPW,), jnp.float32),
          pltpu.SemaphoreType.DMA,
          pltpu.SemaphoreType.DMA,
      ],
  )
  def k(ttab_hbm, ctab_hbm, tgt_hbm, ctx_hbm, out_hbm,
        tidx_v, cidx_v, trows_v, crows_v, out_v, sem0, sem1):
    wid = lax.axis_index("s") * NC + lax.axis_index("c")
    pltpu.sync_copy(tgt_hbm.at[pl.ds(wid * BPW, BPW)], tidx_v)
    for c in range(C):
      pltpu.sync_copy(ctx_hbm.at[pl.ds(c * B + wid * BPW, BPW)],
                      cidx_v.at[pl.ds(c * BPW, BPW)])

    lane = lax.iota(jnp.int32, LANES)
    cmask = [lane == c for c in range(C)]
    store_mask = lane < C
    sems = [sem0, sem1]

    def issue(j):
      buf = j % NBUF
      sem = sems[buf]
      cps = [pltpu.async_copy(
          ttab_hbm.at[tidx_v.at[pl.ds(j * CHUNK, CHUNK)]],
          trows_v.at[buf], sem)]
      for i in range(C):
        cps.append(
            pltpu.async_copy(
                ctab_hbm.at[cidx_v.at[pl.ds(i * BPW + j * CHUNK, CHUNK)]],
                crows_v.at[buf].at[pl.ds(i * CHUNK, CHUNK)], sem))
      return cps

    inflight = [issue(0)]
    for j in range(NCHUNK):
      if j + 1 < NCHUNK:
        inflight.append(issue(j + 1))
      for cp_ in inflight.pop(0):
        cp_.wait()
      buf = j % NBUF

      @pl.loop(0, CHUNK)
      def _(b):
        w = [trows_v[buf, b, pl.ds(LANES * k, LANES)]
             for k in range(E // LANES)]
        dots = jnp.zeros((LANES,), jnp.float32)
        for c in range(C):
          r = c * CHUNK + b
          acc = w[0] * crows_v[buf, r, pl.ds(0, LANES)]
          for k in range(1, E // LANES):
            acc = acc + w[k] * crows_v[buf, r, pl.ds(LANES * k, LANES)]
          dots = jnp.where(cmask[c], jnp.sum(acc), dots)
        plsc.store_scatter(out_v, [lane * BPW + (j * CHUNK + b)], dots,
                           mask=store_mask)

    for c in range(C):
      pltpu.sync_copy(out_v.at[pl.ds(c * BPW, BPW)],
                      out_hbm.at[pl.ds(c * B + wid * BPW, BPW)])

  return k(t_table, c_table, tgt_idx, ctx_idx)


def kernel(target, context, target_table, context_table):
  eye = jnp.eye(E, 2 * E, dtype=jnp.float32)
  t_table = jnp.einsum("ve,ef->vf", target_table, eye,
                       preferred_element_type=jnp.float32)
  c_table = jnp.einsum("ve,ef->vf", context_table, eye,
                       preferred_element_type=jnp.float32)
  tgt_idx = target.reshape(-1).astype(jnp.int32)
  ctx_idx = context.T.reshape(-1).astype(jnp.int32)
  out = _sc_note2vec(t_table, c_table, tgt_idx, ctx_idx)
  return out.reshape(C, B).T
